# baseline (speedup 1.0000x reference)
import dataclasses
import functools

import jax
import jax.numpy as jnp
from jax import lax
from jax.experimental import pallas as pl
from jax.experimental.pallas import tpu as pltpu
from jax.experimental.pallas import tpu_sc as plsc

_B, _L, _D = 1024, 200, 128
_NL = 16
_LP = 208
_NG = _LP // _NL
_NC, _NS = 2, 16
_NW = _NC * _NS
_BPW = _B // _NW
_E = 3 * _D


def _compiler_params():
    cp = pltpu.CompilerParams()
    fields = pltpu.CompilerParams.__dataclass_fields__
    if "needs_layout_passes" in fields:
        cp = dataclasses.replace(cp, needs_layout_passes=False)
    return cp


def _sc_combine(embed, shortlist, weight, bias, attention_weights):
    mesh = plsc.VectorSubcoreMesh(core_axis_name="c", subcore_axis_name="s")

    embed_flat = embed.reshape(-1)
    short_flat = shortlist.reshape(-1)
    att0 = attention_weights[:, 0]
    att1 = attention_weights[:, 1]
    att2 = attention_weights[:, 2]

    @functools.partial(
        pl.kernel,
        out_type=jax.ShapeDtypeStruct((_B * _L,), jnp.float32),
        mesh=mesh,
        compiler_params=_compiler_params(),
        scratch_types=[
            pltpu.VMEM((_BPW * _L + _NL,), jnp.int32),
            pltpu.VMEM((_BPW * _E,), jnp.float32),
            [pltpu.VMEM((_LP, _D), jnp.float32) for _ in range(2)],
            [pltpu.VMEM((_LP,), jnp.float32) for _ in range(2)],
            [pltpu.VMEM((_LP,), jnp.float32) for _ in range(2)],
            [pltpu.VMEM((_LP,), jnp.float32) for _ in range(2)],
            [pltpu.VMEM((_LP,), jnp.float32) for _ in range(2)],
            [pltpu.VMEM((_LP,), jnp.float32) for _ in range(2)],
            [pltpu.SemaphoreType.DMA for _ in range(2)],
            [pltpu.SemaphoreType.DMA for _ in range(2)],
        ],
    )
    def run(embed_hbm, short_hbm, weight_hbm, bias_hbm,
            att0_hbm, att1_hbm, att2_hbm, out_hbm,
            idx_all, emb_all, rows, a0, a1, a2, bb, ob, semg, semo):
        wid = lax.axis_index("s") * _NC + lax.axis_index("c")
        iota = lax.iota(jnp.int32, _NL)
        base_l = wid * (_BPW * _L)

        pltpu.sync_copy(short_hbm.at[pl.ds(base_l, _BPW * _L)],
                        idx_all.at[pl.ds(0, _BPW * _L)])
        pltpu.sync_copy(embed_hbm.at[pl.ds(wid * (_BPW * _E), _BPW * _E)],
                        emb_all)
        idx_all[pl.ds(_BPW * _L, _NL)] = jnp.zeros((_NL,), jnp.int32)

        def gather_descs(j, par):
            sl = idx_all.at[pl.ds(j * _L, _LP)]
            sem = semg[par]
            return [
                (weight_hbm.at[sl], rows[par], sem),
                (att0_hbm.at[sl], a0[par], sem),
                (att1_hbm.at[sl], a1[par], sem),
                (att2_hbm.at[sl], a2[par], sem),
                (bias_hbm.at[sl], bb[par], sem),
            ]

        def issue(j, par):
            for src, dst, sem in gather_descs(j, par):
                pltpu.async_copy(src, dst, sem)

        def wait_gathers(j, par):
            for src, dst, sem in gather_descs(j, par):
                pltpu.make_async_copy(src, dst, sem).wait()

        def out_desc(j, par):
            return (ob[par].at[pl.ds(0, _L)],
                    out_hbm.at[pl.ds(base_l + j * _L, _L)], semo[par])

        def compute(j, par):
            eoff = j * _E
            for g0, ng in ((0, 7), (7, 6)):
                row_idx = [iota + (g0 + g) * _NL for g in range(ng)]
                init = tuple(jnp.zeros((_NL,), jnp.float32)
                             for _ in range(3 * ng))

                def dstep(d, accs, row_idx=row_idx, ng=ng):
                    base = d & (-_NL)
                    rot = (iota + d) & (_NL - 1)
                    col = rot + base
                    ev0 = emb_all[pl.ds(eoff + base, _NL)]
                    ev1 = emb_all[pl.ds(eoff + _D + base, _NL)]
                    ev2 = emb_all[pl.ds(eoff + 2 * _D + base, _NL)]
                    e0 = ev0[rot]
                    e1 = ev1[rot]
                    e2 = ev2[rot]
                    accs = list(accs)
                    for g in range(ng):
                        w = plsc.load_gather(rows[par],
                                             [row_idx[g], col])
                        accs[3 * g] = accs[3 * g] + w * e0
                        accs[3 * g + 1] = accs[3 * g + 1] + w * e1
                        accs[3 * g + 2] = accs[3 * g + 2] + w * e2
                    return tuple(accs)

                accs = lax.fori_loop(0, _D, dstep, init, unroll=2)

                for g in range(ng):
                    gi = g0 + g
                    sl = pl.ds(gi * _NL, _NL)
                    x0 = a0[par][sl]
                    x1 = a1[par][sl]
                    x2 = a2[par][sl]
                    m = jnp.maximum(jnp.maximum(x0, x1), x2)
                    x0 = jnp.exp(x0 - m)
                    x1 = jnp.exp(x1 - m)
                    x2 = jnp.exp(x2 - m)
                    s = x0 + x1 + x2
                    r = (accs[3 * g] * x0 + accs[3 * g + 1] * x1
                         + accs[3 * g + 2] * x2) / s + bb[par][sl]
                    ob[par][sl] = r

        issue(0, 0)
        issue(1, 1)

        @pl.loop(0, _BPW // 2)
        def _(t):
            for par in (0, 1):
                j = 2 * t + par
                wait_gathers(j, par)

                @pl.when(j >= 2)
                def _():
                    src, dst, sem = out_desc(j - 2, par)
                    pltpu.make_async_copy(src, dst, sem).wait()

                compute(j, par)
                src, dst, sem = out_desc(j, par)
                pltpu.async_copy(src, dst, sem)

                @pl.when(j + 2 < _BPW)
                def _():
                    issue(j + 2, par)

        for par, j in ((0, _BPW - 2), (1, _BPW - 1)):
            src, dst, sem = out_desc(j, par)
            pltpu.make_async_copy(src, dst, sem).wait()

    out = run(embed_flat, short_flat, weight, bias, att0, att1, att2)
    return out.reshape(_B, _L)


def kernel(embed, shortlist, weight, bias, attention_weights):
    return _sc_combine(embed, shortlist, weight, bias, attention_weights)
